# baseline (speedup 1.0000x reference)
import functools

import jax
import jax.numpy as jnp
from jax import lax
from jax.experimental import pallas as pl
from jax.experimental.pallas import tpu as pltpu
from jax.experimental.pallas import tpu_sc as plsc

_info = plsc.get_sparse_core_info()
_NC, _NS = _info.num_cores, _info.num_subcores
_NW = _NC * _NS

_CHUNK_ROWS = 8
_NBUF = 6
_LEAD = 5


def _make_copy_kernel(seq_len: int, d_model: int):
    rows_per_w = seq_len // _NW
    n_chunks = rows_per_w // _CHUNK_ROWS
    mesh = plsc.VectorSubcoreMesh(core_axis_name="c", subcore_axis_name="s")

    @functools.partial(
        pl.kernel,
        mesh=mesh,
        out_type=jax.ShapeDtypeStruct((seq_len, d_model), jnp.float32),
        scratch_types=(
            [pltpu.VMEM_SHARED((_NS, _NBUF, _CHUNK_ROWS, d_model), jnp.float32)]
            + [pltpu.SemaphoreType.DMA for _ in range(2 * _NBUF)]
        ),
    )
    def copy_rows(table_hbm, out_hbm, *scratch):
        shared = scratch[0]
        lsem = list(scratch[1 : 1 + _NBUF])
        ssem = list(scratch[1 + _NBUF :])
        sid = lax.axis_index("s")
        wid = sid * _NC + lax.axis_index("c")
        base = wid * rows_per_w
        bufs = [shared.at[sid, k] for k in range(_NBUF)]

        def src(i):
            return table_hbm.at[pl.ds(base + i * _CHUNK_ROWS, _CHUNK_ROWS)]

        def dst(i):
            return out_hbm.at[pl.ds(base + i * _CHUNK_ROWS, _CHUNK_ROWS)]

        loads = [None] * n_chunks
        stores = [None] * n_chunks
        for j in range(min(_LEAD, n_chunks)):
            loads[j] = pltpu.async_copy(src(j), bufs[j % _NBUF], lsem[j % _NBUF])
        for i in range(n_chunks):
            b = i % _NBUF
            loads[i].wait()
            stores[i] = pltpu.async_copy(bufs[b], dst(i), ssem[b])
            j = i + _LEAD
            if j < n_chunks:
                if j - _NBUF >= 0:
                    stores[j - _NBUF].wait()
                loads[j] = pltpu.async_copy(src(j), bufs[j % _NBUF], lsem[j % _NBUF])
        for i in range(max(0, n_chunks - _NBUF), n_chunks):
            stores[i].wait()

    return copy_rows


@jax.jit
def kernel(inputs, pos_table):
    seq_len = inputs.shape[1]
    return _make_copy_kernel(seq_len, pos_table.shape[1])(pos_table)


# speedup vs baseline: 1.0313x; 1.0313x over previous
import functools
import types

import jax
import jax.numpy as jnp
from jax import lax
from jax.experimental import pallas as pl
from jax.experimental.pallas import tpu as pltpu
from jax.experimental.pallas import tpu_sc as plsc

_info = plsc.get_sparse_core_info()
_NC, _NS = _info.num_cores, _info.num_subcores
_NW = _NC * _NS

_CHUNK_ROWS = 8
_NBUF = 4
_SPLIT = 8


def _prime(p):
    for j in range(min(p.lead, p.n)):
        p.loads[j] = pltpu.async_copy(p.src(j), p.bufs[j % p.nbuf], p.lsem[j % p.nbuf])


def _step(p, i):
    if i >= p.n:
        return
    b = i % p.nbuf
    p.loads[i].wait()
    p.stores[i] = pltpu.async_copy(p.bufs[b], p.dst(i), p.ssem[b])
    j = i + p.lead
    if j < p.n:
        if j - p.nbuf >= 0:
            p.stores[j - p.nbuf].wait()
        p.loads[j] = pltpu.async_copy(p.src(j), p.bufs[j % p.nbuf], p.lsem[j % p.nbuf])


def _drain(p):
    for i in range(max(0, p.n - p.nbuf), p.n):
        p.stores[i].wait()


def _make_copy_kernel(seq_len: int, d_model: int):
    rows_per_w = seq_len // _NW
    n_chunks = rows_per_w // _CHUNK_ROWS
    n_a = _SPLIT
    n_b = n_chunks - _SPLIT
    mesh = plsc.VectorSubcoreMesh(core_axis_name="c", subcore_axis_name="s")

    @functools.partial(
        pl.kernel,
        mesh=mesh,
        out_type=jax.ShapeDtypeStruct((seq_len, d_model), jnp.float32),
        scratch_types=(
            [pltpu.VMEM((_CHUNK_ROWS, d_model), jnp.float32) for _ in range(_NBUF)]
            + [pltpu.VMEM_SHARED((_NS, _NBUF, _CHUNK_ROWS, d_model), jnp.float32)]
            + [pltpu.SemaphoreType.DMA for _ in range(4 * _NBUF)]
        ),
    )
    def copy_rows(table_hbm, out_hbm, *scratch):
        vbufs = list(scratch[:_NBUF])
        shared = scratch[_NBUF]
        sems = list(scratch[_NBUF + 1 :])
        sid = lax.axis_index("s")
        wid = sid * _NC + lax.axis_index("c")
        base = wid * rows_per_w

        def src(i):
            return table_hbm.at[pl.ds(base + i * _CHUNK_ROWS, _CHUNK_ROWS)]

        def dst(i):
            return out_hbm.at[pl.ds(base + i * _CHUNK_ROWS, _CHUNK_ROWS)]

        pipe_a = types.SimpleNamespace(
            n=n_a, nbuf=_NBUF, lead=_NBUF - 1,
            bufs=vbufs,
            lsem=sems[:_NBUF], ssem=sems[_NBUF : 2 * _NBUF],
            src=src, dst=dst,
            loads=[None] * n_a, stores=[None] * n_a,
        )
        pipe_b = types.SimpleNamespace(
            n=n_b, nbuf=_NBUF, lead=_NBUF - 1,
            bufs=[shared.at[sid, k] for k in range(_NBUF)],
            lsem=sems[2 * _NBUF : 3 * _NBUF], ssem=sems[3 * _NBUF :],
            src=lambda i: src(n_a + i), dst=lambda i: dst(n_a + i),
            loads=[None] * n_b, stores=[None] * n_b,
        )

        _prime(pipe_a)
        _prime(pipe_b)
        for i in range(max(n_a, n_b)):
            _step(pipe_a, i)
            _step(pipe_b, i)
        _drain(pipe_a)
        _drain(pipe_b)

    return copy_rows


@jax.jit
def kernel(inputs, pos_table):
    seq_len = inputs.shape[1]
    return _make_copy_kernel(seq_len, pos_table.shape[1])(pos_table)


# speedup vs baseline: 1.0456x; 1.0139x over previous
import functools

import jax
import jax.numpy as jnp
from jax import lax
from jax.experimental import pallas as pl
from jax.experimental.pallas import tpu as pltpu
from jax.experimental.pallas import tpu_sc as plsc

_info = plsc.get_sparse_core_info()
_NC, _NS = _info.num_cores, _info.num_subcores
_NW = _NC * _NS

_CHUNK_ROWS = 8
_NBUF = 6
_LEAD = 5


def _make_copy_kernel(seq_len: int, d_model: int):
    rows_per_w = seq_len // _NW
    n_chunks = rows_per_w // _CHUNK_ROWS
    mesh = plsc.VectorSubcoreMesh(core_axis_name="c", subcore_axis_name="s")

    @functools.partial(
        pl.kernel,
        mesh=mesh,
        out_type=jax.ShapeDtypeStruct((seq_len, d_model), jnp.float32),
        scratch_types=(
            [pltpu.VMEM((_CHUNK_ROWS, d_model), jnp.float32) for _ in range(_NBUF)]
            + [pltpu.SemaphoreType.DMA for _ in range(2 * _NBUF)]
        ),
    )
    def copy_rows(table_hbm, out_hbm, *scratch):
        bufs = list(scratch[:_NBUF])
        lsem = list(scratch[_NBUF : 2 * _NBUF])
        ssem = list(scratch[2 * _NBUF :])
        wid = lax.axis_index("s") * _NC + lax.axis_index("c")
        base = wid * rows_per_w

        def src(i):
            return table_hbm.at[pl.ds(base + i * _CHUNK_ROWS, _CHUNK_ROWS)]

        def dst(i):
            return out_hbm.at[pl.ds(base + i * _CHUNK_ROWS, _CHUNK_ROWS)]

        loads = [None] * n_chunks
        stores = [None] * n_chunks
        for j in range(min(_LEAD, n_chunks)):
            loads[j] = pltpu.async_copy(src(j), bufs[j % _NBUF], lsem[j % _NBUF])
        for i in range(n_chunks):
            b = i % _NBUF
            loads[i].wait()
            stores[i] = pltpu.async_copy(bufs[b], dst(i), ssem[b])
            j = i + _LEAD
            if j < n_chunks:
                if j - _NBUF >= 0:
                    stores[j - _NBUF].wait()
                loads[j] = pltpu.async_copy(src(j), bufs[j % _NBUF], lsem[j % _NBUF])
        for i in range(max(0, n_chunks - _NBUF), n_chunks):
            stores[i].wait()

    return copy_rows


@jax.jit
def kernel(inputs, pos_table):
    seq_len = inputs.shape[1]
    return _make_copy_kernel(seq_len, pos_table.shape[1])(pos_table)
